# baseline (speedup 1.0000x reference)
import functools

import jax
import jax.numpy as jnp
from jax import lax
from jax.experimental import pallas as pl
from jax.experimental.pallas import tpu as pltpu
from jax.experimental.pallas import tpu_sc as plsc

N = 10000
E = 320000
R = 2

NC, NS, L = 2, 16, 16
NW = NC * NS
EPW = E // NW
K = 128
NBLK = 79
NBF = NBLK + 1
ND = N + 8
RPS = N // NS


def _elu(x):
    return jnp.where(x > 0, x, jnp.exp(jnp.minimum(x, 0.0)) - 1.0)


def _sc_edge_layer(t, qkflat, packed):
    dout = t.shape[1]
    mesh = plsc.VectorSubcoreMesh(core_axis_name="c", subcore_axis_name="s")

    @functools.partial(
        pl.kernel,
        mesh=mesh,
        compiler_params=pltpu.CompilerParams(use_tc_tiling_on_sc=False,
                                             needs_layout_passes=False),
        out_type=[
            jax.ShapeDtypeStruct((NC, N, dout), jnp.float32),
            jax.ShapeDtypeStruct((NW, N), jnp.float32),
        ],
        scratch_types=[
            pltpu.VMEM((2 * R * N,), jnp.float32),
            pltpu.VMEM((N + L,), jnp.float32),
            pltpu.VMEM((3, K), jnp.int32),
            pltpu.VMEM((3, K), jnp.int32),
            pltpu.VMEM((K,), jnp.int32),
            pltpu.VMEM((K,), jnp.int32),
            pltpu.VMEM((K,), jnp.float32),
            pltpu.VMEM((K,), jnp.float32),
            pltpu.VMEM((K, dout), jnp.float32),
            pltpu.VMEM((K, dout), jnp.float32),
            pltpu.VMEM_SHARED((ND, dout), jnp.float32),
            pltpu.SemaphoreType.DMA,
            pltpu.SemaphoreType.DMA,
            pltpu.SemaphoreType.DMA,
            pltpu.SemaphoreType.DMA,
            pltpu.SemaphoreType.DMA,
            pltpu.SemaphoreType.DMA,
        ],
    )
    def k(t_hbm, qk_hbm, e_hbm, out_hbm, dp_hbm,
          qk_v, den_v, e0, e1, db0, db1, ex0, ex1, r0, r1, acc_sh,
          gs0, gs1, ss0, ss1, es0, es1):
        cid = lax.axis_index("c")
        sid = lax.axis_index("s")
        wid = sid * NC + cid
        pltpu.async_copy(e_hbm.at[wid, 0], e0, es0)
        pltpu.async_copy(e_hbm.at[wid, 1], e1, es1)
        pltpu.sync_copy(qk_hbm, qk_v)

        z16 = jnp.zeros((L,), jnp.float32)
        zi16 = jnp.zeros((L,), jnp.int32)

        @plsc.parallel_loop(0, N + L, step=L, unroll=8)
        def _(i):
            den_v[pl.ds(i, L)] = z16

        def compute(x, eb, dst_b, ex_b):
            for j in range(0, K, L):
                ia2v = eb[0, pl.ds(j, L)]
                ibv = eb[1, pl.ds(j, L)]
                dv = eb[2, pl.ds(j, L)]
                qn = plsc.load_gather(qk_v, [ia2v])
                kn = plsc.load_gather(qk_v, [ibv * 2 + 1])
                a = qn + kn
                a = jnp.where(a >= 0.0, a, 0.2 * a)
                exv = jnp.exp(a)
                dst_b[pl.ds(j, L)] = dv
                ex_b[pl.ds(j, L)] = exv
                plsc.addupdate_scatter(den_v, [dv], exv)

        lanes = lax.iota(jnp.int32, L)

        def scale(rows, ex_b):
            @plsc.parallel_loop(0, K, step=L, unroll=4)
            def _(j):
                w16 = ex_b[pl.ds(j, L)]
                for u in range(L):
                    wu = jnp.sum(jnp.where(lanes == u, w16, 0.0))
                    for c in range(dout // L):
                        sl = pl.ds(c * L, L)
                        rows[j + u, sl] = rows[j + u, sl] * wu

        def start_gather(eb, rows, sem):
            pltpu.async_copy(t_hbm.at[eb.at[1]], rows, sem)

        def wait_gather(eb, rows, sem):
            pltpu.make_async_copy(t_hbm.at[eb.at[1]], rows, sem).wait()

        def start_scatter(rows, dst_b, sem):
            pltpu.async_copy(rows, acc_sh.at[dst_b], sem, add=True)

        def wait_scatter(rows, dst_b, sem):
            pltpu.make_async_copy(rows, acc_sh.at[dst_b], sem).wait()

        def wait_prefetch(x, eb, sem):
            pltpu.make_async_copy(e_hbm.at[wid, x], eb, sem).wait()

        wait_prefetch(0, e0, es0)
        compute(0, e0, db0, ex0)
        start_gather(e0, r0, gs0)

        @pl.loop(0, K)
        def _(i):
            for c in range(dout // L):
                r1[i, pl.ds(c * L, L)] = z16
        for j in range(0, K, L):
            db1[pl.ds(j, L)] = zi16

        @pl.loop(0, RPS - K + 1, step=K)
        def _(j):
            pltpu.sync_copy(r1, acc_sh.at[pl.ds(sid * RPS + j, K)])
        rem = RPS % K
        pltpu.sync_copy(r1.at[pl.ds(0, rem)],
                        acc_sh.at[pl.ds(sid * RPS + RPS - rem, rem)])
        plsc.subcore_barrier()
        start_scatter(r1, db1, ss1)

        @pl.loop(0, (NBLK - 1) // 2)
        def _(i):
            x = 2 * i
            wait_scatter(r1, db1, ss1)
            wait_prefetch(x + 1, e1, es1)
            compute(x + 1, e1, db1, ex1)
            start_gather(e1, r1, gs1)
            wait_gather(e0, r0, gs0)
            pltpu.async_copy(e_hbm.at[wid, x + 2], e0, es0)
            scale(r0, ex0)
            start_scatter(r0, db0, ss0)

            wait_scatter(r0, db0, ss0)
            wait_prefetch(x + 2, e0, es0)
            compute(x + 2, e0, db0, ex0)
            start_gather(e0, r0, gs0)
            wait_gather(e1, r1, gs1)
            pltpu.async_copy(e_hbm.at[wid, x + 3], e1, es1)
            scale(r1, ex1)
            start_scatter(r1, db1, ss1)

        wait_scatter(r1, db1, ss1)
        wait_prefetch(NBF - 1, e1, es1)
        wait_gather(e0, r0, gs0)
        scale(r0, ex0)
        start_scatter(r0, db0, ss0)
        wait_scatter(r0, db0, ss0)

        pltpu.sync_copy(den_v.at[pl.ds(0, N)], dp_hbm.at[wid])
        plsc.subcore_barrier()
        pltpu.sync_copy(acc_sh.at[pl.ds(sid * RPS, RPS)],
                        out_hbm.at[cid, pl.ds(sid * RPS, RPS)])

    return k(t, qkflat, packed)


def _transform_x_body(x_ref, w_ref, qk_ref, t_ref, qkn_ref):
    t = jnp.dot(x_ref[...], w_ref[0], preferred_element_type=jnp.float32)
    t_ref[...] = t
    qkn_ref[...] = jnp.dot(t, qk_ref[...], preferred_element_type=jnp.float32)


def _transform_parts_body(p_ref, dp_ref, w_ref, qk_ref, t_ref, qkn_ref):
    den = jnp.sum(dp_ref[...], axis=0)
    dinv = 1.0 / (den + 1e-16)
    x = _elu((p_ref[0] + p_ref[1]) * dinv[:, None])
    t = jnp.dot(x, w_ref[0], preferred_element_type=jnp.float32)
    t_ref[...] = t
    qkn_ref[...] = jnp.dot(t, qk_ref[...], preferred_element_type=jnp.float32)


def _transform(x_or_parts, dparts, W, q, k):
    din, dout = W.shape[1], W.shape[2]
    qk = jnp.concatenate([q, k], axis=1)
    if dparts is None:
        body = _transform_x_body
        in_specs = [pl.BlockSpec((N, din), lambda r: (0, 0))]
        args = (x_or_parts,)
    else:
        body = _transform_parts_body
        in_specs = [
            pl.BlockSpec((2, N, din), lambda r: (0, 0, 0)),
            pl.BlockSpec((NW, N), lambda r: (0, 0)),
        ]
        args = (x_or_parts, dparts)
    return pl.pallas_call(
        body,
        grid=(R,),
        in_specs=in_specs + [
            pl.BlockSpec((1, din, dout), lambda r: (r, 0, 0)),
            pl.BlockSpec((dout, 2), lambda r: (0, 0)),
        ],
        out_specs=[
            pl.BlockSpec((N, dout), lambda r: (r, 0)),
            pl.BlockSpec((N, 2), lambda r: (r, 0)),
        ],
        out_shape=[
            jax.ShapeDtypeStruct((R * N, dout), jnp.float32),
            jax.ShapeDtypeStruct((R * N, 2), jnp.float32),
        ],
    )(*args, W, qk)


def _decoder_body(p_ref, dp_ref, dw1_ref, db1_ref, dw2_ref, db2_ref,
                  dw3_ref, db3_ref, h3_ref, out_ref):
    den = jnp.sum(dp_ref[...], axis=0)
    dinv = 1.0 / (den + 1e-16)
    h3 = _elu((p_ref[0] + p_ref[1]) * dinv[:, None])
    h3_ref[...] = h3
    z = jnp.maximum(jnp.dot(h3, dw1_ref[...], preferred_element_type=jnp.float32)
                    + db1_ref[...], 0.0)
    z = jnp.maximum(jnp.dot(z, dw2_ref[...], preferred_element_type=jnp.float32)
                    + db2_ref[...], 0.0)
    out_ref[...] = jnp.dot(z, dw3_ref[...], preferred_element_type=jnp.float32) \
        + db3_ref[...]


def _decoder(parts3, dparts3, dw1, db1, dw2, db2, dw3, db3):
    d3 = parts3.shape[-1]
    return pl.pallas_call(
        _decoder_body,
        out_shape=[
            jax.ShapeDtypeStruct((N, d3), jnp.float32),
            jax.ShapeDtypeStruct((N, 128), jnp.float32),
        ],
    )(parts3, dparts3, dw1, db1.reshape(1, -1), dw2, db2.reshape(1, -1),
      dw3, db3.reshape(1, -1))


def kernel(features, edge_index, edge_type, W1, q1, k1, W2, q2, k2, W3, q3, k3,
           dw1, db1, dw2, db2, dw3, db3):
    src = edge_index[0]
    dst = edge_index[1]
    ia2 = (edge_type * N + dst) * 2
    ib = edge_type * N + src

    def pad_pack(a, padval):
        a = a.reshape(NW, EPW)
        a = jnp.pad(a, ((0, 0), (0, NBF * K - EPW)), constant_values=padval)
        return a.reshape(NW, NBF, 1, K)

    packed = jnp.concatenate(
        [pad_pack(ia2, 0), pad_pack(ib, 0), pad_pack(dst, N)], axis=2)

    T1, qkn1 = _transform(features, None, W1, q1, k1)
    p1, dp1 = _sc_edge_layer(T1, qkn1.reshape(-1), packed)
    T2, qkn2 = _transform(p1, dp1, W2, q2, k2)
    p2, dp2 = _sc_edge_layer(T2, qkn2.reshape(-1), packed)
    T3, qkn3 = _transform(p2, dp2, W3, q3, k3)
    p3, dp3 = _sc_edge_layer(T3, qkn3.reshape(-1), packed)
    h3, out = _decoder(p3, dp3, dw1, db1, dw2, db2, dw3, db3)
    return (h3, out)


# speedup vs baseline: 1.2654x; 1.2654x over previous
import functools

import jax
import jax.numpy as jnp
from jax import lax
from jax.experimental import pallas as pl
from jax.experimental.pallas import tpu as pltpu
from jax.experimental.pallas import tpu_sc as plsc

N = 10000
E = 320000
R = 2

NC, NS, L = 2, 16, 16
NW = NC * NS
EPW = E // NW
K = 80
NBLK = EPW // K
RPS = N // NS


def _elu(x):
    return jnp.where(x > 0, x, jnp.exp(jnp.minimum(x, 0.0)) - 1.0)


def _sc_edge_layer(t, qkflat, et3, src3, dst3):
    dout = t.shape[1]
    mesh = plsc.VectorSubcoreMesh(core_axis_name="c", subcore_axis_name="s")

    @functools.partial(
        pl.kernel,
        mesh=mesh,
        compiler_params=pltpu.CompilerParams(use_tc_tiling_on_sc=False,
                                             needs_layout_passes=False),
        out_type=[
            jax.ShapeDtypeStruct((NC, N, dout), jnp.float32),
            jax.ShapeDtypeStruct((NW, N), jnp.float32),
        ],
        scratch_types=[
            pltpu.VMEM((NBLK, K), jnp.int32),
            pltpu.VMEM((NBLK, K), jnp.int32),
            pltpu.VMEM((NBLK, K), jnp.int32),
            pltpu.VMEM((2 * R * N,), jnp.float32),
            pltpu.VMEM((N,), jnp.float32),
            pltpu.VMEM((K,), jnp.int32),
            pltpu.VMEM((K,), jnp.int32),
            pltpu.VMEM((K,), jnp.float32),
            pltpu.VMEM((K,), jnp.int32),
            pltpu.VMEM((K,), jnp.int32),
            pltpu.VMEM((K,), jnp.float32),
            pltpu.VMEM((K, dout), jnp.float32),
            pltpu.VMEM((K, dout), jnp.float32),
            pltpu.VMEM_SHARED((N, dout), jnp.float32),
            pltpu.SemaphoreType.DMA,
            pltpu.SemaphoreType.DMA,
            pltpu.SemaphoreType.DMA,
            pltpu.SemaphoreType.DMA,
        ],
    )
    def k(t_hbm, qk_hbm, et_hbm, src_hbm, dst_hbm, out_hbm, dp_hbm,
          et_v, src_v, dst_v, qk_v, den_v,
          ib0, db0, ex0, ib1, db1, ex1, r0, r1, acc_sh,
          gs0, gs1, ss0, ss1):
        cid = lax.axis_index("c")
        sid = lax.axis_index("s")
        wid = sid * NC + cid
        pltpu.sync_copy(et_hbm.at[wid], et_v)
        pltpu.sync_copy(src_hbm.at[wid], src_v)
        pltpu.sync_copy(dst_hbm.at[wid], dst_v)
        pltpu.sync_copy(qk_hbm, qk_v)

        z16 = jnp.zeros((L,), jnp.float32)
        zi16 = jnp.zeros((L,), jnp.int32)

        @plsc.parallel_loop(0, N, step=L, unroll=8)
        def _(i):
            den_v[pl.ds(i, L)] = z16

        def compute(x, ib_b, dst_b, ex_b):
            for j in range(0, K, L):
                e16 = et_v[x, pl.ds(j, L)]
                s16 = src_v[x, pl.ds(j, L)]
                d16 = dst_v[x, pl.ds(j, L)]
                ib16 = e16 * N + s16
                ia2 = (e16 * N + d16) * 2
                ib2 = ib16 * 2 + 1
                qn = plsc.load_gather(qk_v, [ia2])
                kn = plsc.load_gather(qk_v, [ib2])
                a = qn + kn
                a = jnp.where(a >= 0.0, a, 0.2 * a)
                exv = jnp.exp(a)
                ib_b[pl.ds(j, L)] = ib16
                dst_b[pl.ds(j, L)] = d16
                ex_b[pl.ds(j, L)] = exv
                plsc.addupdate_scatter(den_v, [d16], exv)

        lanes = lax.iota(jnp.int32, L)

        def scale(rows, ex_b):
            @plsc.parallel_loop(0, K, step=L, unroll=5)
            def _(j):
                w16 = ex_b[pl.ds(j, L)]
                for u in range(L):
                    wu = jnp.sum(jnp.where(lanes == u, w16, 0.0))
                    for c in range(dout // L):
                        sl = pl.ds(c * L, L)
                        rows[j + u, sl] = rows[j + u, sl] * wu

        def start_gather(ib_b, rows, sem):
            pltpu.async_copy(t_hbm.at[ib_b], rows, sem)

        def wait_gather(ib_b, rows, sem):
            pltpu.make_async_copy(t_hbm.at[ib_b], rows, sem).wait()

        def start_scatter(rows, dst_b, sem):
            pltpu.async_copy(rows, acc_sh.at[dst_b], sem, add=True)

        def wait_scatter(rows, dst_b, sem):
            pltpu.make_async_copy(rows, acc_sh.at[dst_b], sem).wait()

        compute(0, ib0, db0, ex0)
        start_gather(ib0, r0, gs0)

        @pl.loop(0, K)
        def _(i):
            for c in range(dout // L):
                r1[i, pl.ds(c * L, L)] = z16
        for j in range(0, K, L):
            db1[pl.ds(j, L)] = zi16

        @pl.loop(0, RPS - K + 1, step=K)
        def _(j):
            pltpu.sync_copy(r1, acc_sh.at[pl.ds(sid * RPS + j, K)])
        rem = RPS % K
        pltpu.sync_copy(r1.at[pl.ds(0, rem)],
                        acc_sh.at[pl.ds(sid * RPS + RPS - rem, rem)])
        plsc.subcore_barrier()
        start_scatter(r1, db1, ss1)

        @pl.loop(0, (NBLK - 1) // 2)
        def _(i):
            x = 2 * i
            wait_scatter(r1, db1, ss1)
            compute(x + 1, ib1, db1, ex1)
            start_gather(ib1, r1, gs1)
            wait_gather(ib0, r0, gs0)
            scale(r0, ex0)
            start_scatter(r0, db0, ss0)

            wait_scatter(r0, db0, ss0)
            compute(x + 2, ib0, db0, ex0)
            start_gather(ib0, r0, gs0)
            wait_gather(ib1, r1, gs1)
            scale(r1, ex1)
            start_scatter(r1, db1, ss1)

        wait_scatter(r1, db1, ss1)
        wait_gather(ib0, r0, gs0)
        scale(r0, ex0)
        start_scatter(r0, db0, ss0)
        wait_scatter(r0, db0, ss0)

        pltpu.sync_copy(den_v, dp_hbm.at[wid])
        plsc.subcore_barrier()
        pltpu.sync_copy(acc_sh.at[pl.ds(sid * RPS, RPS)],
                        out_hbm.at[cid, pl.ds(sid * RPS, RPS)])

    return k(t, qkflat, et3, src3, dst3)


def _transform_x_body(x_ref, w_ref, qk_ref, t_ref, qkn_ref):
    t = jnp.dot(x_ref[...], w_ref[0], preferred_element_type=jnp.float32)
    t_ref[...] = t
    qkn_ref[...] = jnp.dot(t, qk_ref[...], preferred_element_type=jnp.float32)


def _transform_parts_body(p_ref, dp_ref, w_ref, qk_ref, t_ref, qkn_ref):
    den = jnp.sum(dp_ref[...], axis=0)
    dinv = 1.0 / (den + 1e-16)
    x = _elu((p_ref[0] + p_ref[1]) * dinv[:, None])
    t = jnp.dot(x, w_ref[0], preferred_element_type=jnp.float32)
    t_ref[...] = t
    qkn_ref[...] = jnp.dot(t, qk_ref[...], preferred_element_type=jnp.float32)


def _transform(x_or_parts, dparts, W, q, k):
    din, dout = W.shape[1], W.shape[2]
    qk = jnp.concatenate([q, k], axis=1)
    if dparts is None:
        body = _transform_x_body
        in_specs = [pl.BlockSpec((N, din), lambda r: (0, 0))]
        args = (x_or_parts,)
    else:
        body = _transform_parts_body
        in_specs = [
            pl.BlockSpec((2, N, din), lambda r: (0, 0, 0)),
            pl.BlockSpec((NW, N), lambda r: (0, 0)),
        ]
        args = (x_or_parts, dparts)
    return pl.pallas_call(
        body,
        grid=(R,),
        in_specs=in_specs + [
            pl.BlockSpec((1, din, dout), lambda r: (r, 0, 0)),
            pl.BlockSpec((dout, 2), lambda r: (0, 0)),
        ],
        out_specs=[
            pl.BlockSpec((N, dout), lambda r: (r, 0)),
            pl.BlockSpec((N, 2), lambda r: (r, 0)),
        ],
        out_shape=[
            jax.ShapeDtypeStruct((R * N, dout), jnp.float32),
            jax.ShapeDtypeStruct((R * N, 2), jnp.float32),
        ],
    )(*args, W, qk)


def _decoder_body(p_ref, dp_ref, dw1_ref, db1_ref, dw2_ref, db2_ref,
                  dw3_ref, db3_ref, h3_ref, out_ref):
    den = jnp.sum(dp_ref[...], axis=0)
    dinv = 1.0 / (den + 1e-16)
    h3 = _elu((p_ref[0] + p_ref[1]) * dinv[:, None])
    h3_ref[...] = h3
    z = jnp.maximum(jnp.dot(h3, dw1_ref[...], preferred_element_type=jnp.float32)
                    + db1_ref[...], 0.0)
    z = jnp.maximum(jnp.dot(z, dw2_ref[...], preferred_element_type=jnp.float32)
                    + db2_ref[...], 0.0)
    out_ref[...] = jnp.dot(z, dw3_ref[...], preferred_element_type=jnp.float32) \
        + db3_ref[...]


def _decoder(parts3, dparts3, dw1, db1, dw2, db2, dw3, db3):
    d3 = parts3.shape[-1]
    return pl.pallas_call(
        _decoder_body,
        out_shape=[
            jax.ShapeDtypeStruct((N, d3), jnp.float32),
            jax.ShapeDtypeStruct((N, 128), jnp.float32),
        ],
    )(parts3, dparts3, dw1, db1.reshape(1, -1), dw2, db2.reshape(1, -1),
      dw3, db3.reshape(1, -1))


def kernel(features, edge_index, edge_type, W1, q1, k1, W2, q2, k2, W3, q3, k3,
           dw1, db1, dw2, db2, dw3, db3):
    et3 = edge_type.reshape(NW, NBLK, K)
    src3 = edge_index[0].reshape(NW, NBLK, K)
    dst3 = edge_index[1].reshape(NW, NBLK, K)

    T1, qkn1 = _transform(features, None, W1, q1, k1)
    p1, dp1 = _sc_edge_layer(T1, qkn1.reshape(-1), et3, src3, dst3)
    T2, qkn2 = _transform(p1, dp1, W2, q2, k2)
    p2, dp2 = _sc_edge_layer(T2, qkn2.reshape(-1), et3, src3, dst3)
    T3, qkn3 = _transform(p2, dp2, W3, q3, k3)
    p3, dp3 = _sc_edge_layer(T3, qkn3.reshape(-1), et3, src3, dst3)
    h3, out = _decoder(p3, dp3, dw1, db1, dw2, db2, dw3, db3)
    return (h3, out)
